# baseline (speedup 1.0000x reference)
import jax
import jax.numpy as jnp
from jax import lax
from jax.experimental import pallas as pl
from jax.experimental.pallas import tpu as pltpu
from jax.experimental.pallas import tpu_sc as plsc

N_NODES = 10000
N_EDGES = 320000
D_FEAT = 128
NW = 32
EDGES_PER_W = N_EDGES // NW
CHUNK = 80
NCHUNKS = EDGES_PER_W // CHUNK
NBUF = 4
LANES = 16
GROUPS = CHUNK // LANES


def _score_body(h_hbm, src_hbm, dst_hbm, out_hbm, h_sh, sidx, didx,
                u0, u1, u2, u3, v0, v1, v2, v3, scores, gsem):
    sid = lax.axis_index("s")
    wid = sid * 2 + lax.axis_index("c")
    ebase = wid * EDGES_PER_W

    @pl.when(sid == 0)
    def _():
        pltpu.sync_copy(h_hbm, h_sh)

    pltpu.sync_copy(src_hbm.at[pl.ds(ebase, EDGES_PER_W)], sidx)
    pltpu.sync_copy(dst_hbm.at[pl.ds(ebase, EDGES_PER_W)], didx)
    plsc.subcore_barrier()
    lane = lax.iota(jnp.int32, LANES)
    U = (u0, u1, u2, u3)
    V = (v0, v1, v2, v3)

    def issue(c, p):
        off = c * CHUNK
        pltpu.async_copy(h_sh.at[sidx.at[pl.ds(off, CHUNK)]], U[p], gsem)
        pltpu.async_copy(h_sh.at[didx.at[pl.ds(off, CHUNK)]], V[p], gsem)

    def wait_gather(p):
        pltpu.make_async_copy(
            h_sh.at[sidx.at[pl.ds(0, CHUNK)]], U[p], gsem).wait()
        pltpu.make_async_copy(
            h_sh.at[didx.at[pl.ds(0, CHUNK)]], V[p], gsem).wait()

    def compute(c, p):
        wait_gather(p)
        urows = U[p]
        vrows = V[p]

        def group_body(g, carry2):
            svec = jnp.zeros((LANES,), jnp.float32)
            for k in range(LANES):
                e = g * LANES + k
                acc = jnp.zeros((LANES,), jnp.float32)
                for j in range(D_FEAT // 32):
                    uj = plsc.bitcast(urows[e, pl.ds(j * 16, 16)],
                                      jnp.bfloat16)
                    vj = plsc.bitcast(vrows[e, pl.ds(j * 16, 16)],
                                      jnp.bfloat16)
                    pa, pb = plsc.unpack(uj * vj,
                                         format=plsc.PackFormat.INTERLEAVED)
                    acc = acc + pa + pb
                svec = jnp.where(lane == k, jnp.sum(acc), svec)
            scores[pl.ds(c * CHUNK + g * LANES, LANES)] = svec
            return carry2

        lax.fori_loop(0, GROUPS, group_body, 0, unroll=False)

    for b in range(NBUF - 1):
        issue(b, b)

    def ring_body(i, carry):
        for b in range(NBUF):
            c = i * NBUF + b
            p_ahead = (b + NBUF - 1) % NBUF

            @pl.when(c + NBUF - 1 < NCHUNKS)
            def _():
                issue(c + NBUF - 1, p_ahead)

            compute(c, b)
        return carry

    lax.fori_loop(0, NCHUNKS // NBUF, ring_body, 0, unroll=False)
    compute(NCHUNKS - 1, (NCHUNKS - 1) % NBUF)

    pltpu.sync_copy(scores,
                    out_hbm.at[pl.ds(ebase, EDGES_PER_W)])


@jax.jit
def _scores(h_i32, src, dst):
    mesh = plsc.VectorSubcoreMesh(core_axis_name="c", subcore_axis_name="s")
    return pl.kernel(
        _score_body,
        out_type=jax.ShapeDtypeStruct((N_EDGES,), jnp.float32),
        mesh=mesh,
        compiler_params=pltpu.CompilerParams(
            needs_layout_passes=False, use_tc_tiling_on_sc=False),
        scratch_types=[
            pltpu.VMEM_SHARED((N_NODES, D_FEAT // 2), jnp.int32),
            pltpu.VMEM((EDGES_PER_W,), jnp.int32),
            pltpu.VMEM((EDGES_PER_W,), jnp.int32),
            pltpu.VMEM((CHUNK, D_FEAT // 2), jnp.int32),
            pltpu.VMEM((CHUNK, D_FEAT // 2), jnp.int32),
            pltpu.VMEM((CHUNK, D_FEAT // 2), jnp.int32),
            pltpu.VMEM((CHUNK, D_FEAT // 2), jnp.int32),
            pltpu.VMEM((CHUNK, D_FEAT // 2), jnp.int32),
            pltpu.VMEM((CHUNK, D_FEAT // 2), jnp.int32),
            pltpu.VMEM((CHUNK, D_FEAT // 2), jnp.int32),
            pltpu.VMEM((CHUNK, D_FEAT // 2), jnp.int32),
            pltpu.VMEM((EDGES_PER_W,), jnp.float32),
            pltpu.SemaphoreType.DMA,
        ],
    )(h_i32, src, dst)


def kernel(h, edge_index):
    h_bf = h.astype(jnp.bfloat16)
    h_i32 = lax.bitcast_convert_type(
        h_bf.reshape(N_NODES, D_FEAT // 2, 2), jnp.int32)
    src = edge_index[0]
    dst = edge_index[1]
    return _scores(h_i32, src, dst)[:, None]


# speedup vs baseline: 1.2615x; 1.2615x over previous
import jax
import jax.numpy as jnp
from jax import lax
from jax.experimental import pallas as pl
from jax.experimental.pallas import tpu as pltpu
from jax.experimental.pallas import tpu_sc as plsc

N_NODES = 10000
N_EDGES = 320000
D_FEAT = 128
NW = 32
EDGES_PER_W = N_EDGES // NW
CHUNK = 256
NCHUNKS = EDGES_PER_W // CHUNK
TAIL = EDGES_PER_W - NCHUNKS * CHUNK
NPAIRS = (NCHUNKS - 1) // 2
LANES = 16
GROUPS = CHUNK // LANES


def _dot_group(urows, vrows, lane, g):
    svec = jnp.zeros((LANES,), jnp.float32)
    for k in range(LANES):
        e = g * LANES + k
        acc = jnp.zeros((LANES,), jnp.float32)
        for j in range(D_FEAT // 32):
            uj = plsc.bitcast(urows[e, pl.ds(j * 16, 16)], jnp.bfloat16)
            vj = plsc.bitcast(vrows[e, pl.ds(j * 16, 16)], jnp.bfloat16)
            pa, pb = plsc.unpack(uj * vj, format=plsc.PackFormat.INTERLEAVED)
            acc = acc + pa + pb
        svec = jnp.where(lane == k, jnp.sum(acc), svec)
    return svec


def _score_body(h_hbm, src_hbm, dst_hbm, out_hbm, h_sh, sidx, didx,
                u0, u1, v0, v1, s0, s1, ut, vt, st, gsem, w0, w1):
    sid = lax.axis_index("s")
    wid = sid * 2 + lax.axis_index("c")
    ebase = wid * EDGES_PER_W

    @pl.when(sid == 0)
    def _():
        pltpu.sync_copy(h_hbm, h_sh)

    pltpu.sync_copy(src_hbm.at[pl.ds(ebase, EDGES_PER_W)], sidx)
    pltpu.sync_copy(dst_hbm.at[pl.ds(ebase, EDGES_PER_W)], didx)
    plsc.subcore_barrier()
    lane = lax.iota(jnp.int32, LANES)
    U = (u0, u1)
    V = (v0, v1)
    S = (s0, s1)
    W = (w0, w1)

    def issue(c, p):
        off = c * CHUNK
        pltpu.async_copy(h_sh.at[sidx.at[pl.ds(off, CHUNK)]], U[p], gsem)
        pltpu.async_copy(h_sh.at[didx.at[pl.ds(off, CHUNK)]], V[p], gsem)

    def wait_gather(p):
        pltpu.make_async_copy(
            h_sh.at[sidx.at[pl.ds(0, CHUNK)]], U[p], gsem).wait()
        pltpu.make_async_copy(
            h_sh.at[didx.at[pl.ds(0, CHUNK)]], V[p], gsem).wait()

    def drain_write(p):
        pltpu.make_async_copy(
            S[p], out_hbm.at[pl.ds(ebase, CHUNK)], W[p]).wait()

    def compute(c, p):
        wait_gather(p)
        urows = U[p]
        vrows = V[p]
        scores = S[p]

        def group_body(g, carry2):
            scores[pl.ds(g * LANES, LANES)] = _dot_group(urows, vrows, lane, g)
            return carry2

        lax.fori_loop(0, GROUPS, group_body, 0, unroll=False)
        pltpu.async_copy(scores, out_hbm.at[pl.ds(ebase + c * CHUNK, CHUNK)],
                         W[p])

    issue(0, 0)

    def pair_body(i, carry):
        issue(2 * i + 1, 1)

        @pl.when(i > 0)
        def _():
            drain_write(0)

        compute(2 * i, 0)
        issue(2 * i + 2, 0)

        @pl.when(i > 0)
        def _():
            drain_write(1)

        compute(2 * i + 1, 1)
        return carry

    lax.fori_loop(0, NPAIRS, pair_body, 0, unroll=False)

    if NCHUNKS % 2 == 1:
        drain_write(0)
        compute(NCHUNKS - 1, 0)
    else:
        issue(NCHUNKS - 1, 1)
        drain_write(0)
        compute(NCHUNKS - 2, 0)
        drain_write(1)
        compute(NCHUNKS - 1, 1)
    drain_write(1)
    drain_write(0)

    toff = NCHUNKS * CHUNK
    pltpu.sync_copy(h_sh.at[sidx.at[pl.ds(toff, TAIL)]], ut)
    pltpu.sync_copy(h_sh.at[didx.at[pl.ds(toff, TAIL)]], vt)
    st[pl.ds(0, LANES)] = _dot_group(ut, vt, lane, 0)
    pltpu.sync_copy(st, out_hbm.at[pl.ds(ebase + toff, TAIL)])


@jax.jit
def _scores(h_i32, src, dst):
    mesh = plsc.VectorSubcoreMesh(core_axis_name="c", subcore_axis_name="s")
    return pl.kernel(
        _score_body,
        out_type=jax.ShapeDtypeStruct((N_EDGES,), jnp.float32),
        mesh=mesh,
        compiler_params=pltpu.CompilerParams(
            needs_layout_passes=False, use_tc_tiling_on_sc=False),
        scratch_types=[
            pltpu.VMEM_SHARED((N_NODES, D_FEAT // 2), jnp.int32),
            pltpu.VMEM((EDGES_PER_W,), jnp.int32),
            pltpu.VMEM((EDGES_PER_W,), jnp.int32),
            pltpu.VMEM((CHUNK, D_FEAT // 2), jnp.int32),
            pltpu.VMEM((CHUNK, D_FEAT // 2), jnp.int32),
            pltpu.VMEM((CHUNK, D_FEAT // 2), jnp.int32),
            pltpu.VMEM((CHUNK, D_FEAT // 2), jnp.int32),
            pltpu.VMEM((CHUNK,), jnp.float32),
            pltpu.VMEM((CHUNK,), jnp.float32),
            pltpu.VMEM((TAIL, D_FEAT // 2), jnp.int32),
            pltpu.VMEM((TAIL, D_FEAT // 2), jnp.int32),
            pltpu.VMEM((TAIL,), jnp.float32),
            pltpu.SemaphoreType.DMA,
            pltpu.SemaphoreType.DMA,
            pltpu.SemaphoreType.DMA,
        ],
    )(h_i32, src, dst)


def kernel(h, edge_index):
    h_bf = h.astype(jnp.bfloat16)
    h_i32 = lax.bitcast_convert_type(
        h_bf.reshape(N_NODES, D_FEAT // 2, 2), jnp.int32)
    src = edge_index[0]
    dst = edge_index[1]
    return _scores(h_i32, src, dst)[:, None]
